# baseline (speedup 1.0000x reference)
import jax
import jax.numpy as jnp
from jax.experimental import pallas as pl
from jax.experimental.pallas import tpu as pltpu


def _rewire_kernel(ei_ref, ea_ref, ei_out, ea_out, sem_ei, sem_ea):
    cp_ei = pltpu.make_async_copy(ei_ref, ei_out, sem_ei)
    cp_ea = pltpu.make_async_copy(ea_ref, ea_out, sem_ea)
    cp_ei.start()
    cp_ea.start()
    cp_ei.wait()
    cp_ea.wait()


def kernel(edge_index, edge_attr, num_nodes):
    del num_nodes
    out = pl.pallas_call(
        _rewire_kernel,
        in_specs=[
            pl.BlockSpec(memory_space=pl.ANY),
            pl.BlockSpec(memory_space=pl.ANY),
        ],
        out_specs=(
            pl.BlockSpec(memory_space=pl.ANY),
            pl.BlockSpec(memory_space=pl.ANY),
        ),
        out_shape=(
            jax.ShapeDtypeStruct(edge_index.shape, edge_index.dtype),
            jax.ShapeDtypeStruct(edge_attr.shape, edge_attr.dtype),
        ),
        scratch_shapes=[pltpu.SemaphoreType.DMA, pltpu.SemaphoreType.DMA],
    )(edge_index, edge_attr)
    return out
